# Initial kernel scaffold; baseline (speedup 1.0000x reference)
import jax
import jax.numpy as jnp
from jax.experimental import pallas as pl


def kernel(x, table, W1, b1, W2, b2):
    raise NotImplementedError("write your pallas kernel here")


# speedup vs baseline: 4.1109x; 4.1109x over previous
import functools

import jax
import jax.numpy as jnp
from jax import lax
from jax.experimental import pallas as pl
from jax.experimental.pallas import tpu as pltpu
from jax.experimental.pallas import tpu_sc as plsc

_VOCAB = 1000000
_EMB = 64
_HIDDEN = 256
_B = 16384
_L = 50
_DENSE = 64

_NC = 2
_NS = 16
_NW = _NC * _NS
_BPW = _B // _NW
_CH = 2
_NCHUNK = _BPW // _CH
_LANES = 16


def _sc_emb_body(idx_hbm, table_hbm, out_hbm, idx_v, rows_v, out_v, sem0, sem1):
    wid = lax.axis_index("s") * _NC + lax.axis_index("c")
    pltpu.sync_copy(idx_hbm.at[pl.ds(wid * _NCHUNK, _NCHUNK)], idx_v)
    sems = (sem0, sem1)
    pltpu.async_copy(table_hbm.at[idx_v.at[0]], rows_v.at[0], sem0)

    @pl.loop(0, _NCHUNK, step=2)
    def body(g):
        for b in range(2):
            i = g + b
            nxt = i + 1

            @pl.when(nxt < _NCHUNK)
            def _():
                pltpu.async_copy(
                    table_hbm.at[idx_v.at[nxt]], rows_v.at[1 - b], sems[1 - b])

            pltpu.make_async_copy(
                table_hbm.at[idx_v.at[i]], rows_v.at[b], sems[b]).wait()
            for r in range(_CH):
                for jj in range(0, _EMB // _LANES, 2):
                    accs = [rows_v[b, r * _L, pl.ds((jj + j) * _LANES, _LANES)]
                            for j in range(2)]
                    for l in range(1, _L):
                        for j in range(2):
                            accs[j] = accs[j] + rows_v[b, r * _L + l,
                                                       pl.ds((jj + j) * _LANES,
                                                             _LANES)]
                    for j in range(2):
                        out_v[i * _CH + r,
                              pl.ds((jj + j) * _LANES, _LANES)] = accs[j]

    pltpu.sync_copy(out_v, out_hbm.at[pl.ds(wid * _BPW, _BPW)])


@functools.partial(jax.jit, static_argnames=())
def _sc_emb(idx2, table):
    mesh = plsc.VectorSubcoreMesh(core_axis_name="c", subcore_axis_name="s")
    f = functools.partial(
        pl.kernel,
        out_type=jax.ShapeDtypeStruct((_B, _EMB), jnp.float32),
        mesh=mesh,
        scratch_types=[
            pltpu.VMEM((_NCHUNK, _CH * _L), jnp.int32),
            pltpu.VMEM((2, _CH * _L, _EMB), jnp.float32),
            pltpu.VMEM((_BPW, _EMB), jnp.float32),
            pltpu.SemaphoreType.DMA,
            pltpu.SemaphoreType.DMA,
        ],
        compiler_params=pltpu.CompilerParams(use_tc_tiling_on_sc=False),
    )(_sc_emb_body)
    return f(idx2, table)


_TMAIN = 999936
_TNL = 7936
_TSTEPS = _TMAIN // _TNL


def _tr_body(tT_hbm, o_ref, scr, sem0, sem1):
    i = pl.program_id(0)
    sems = (sem0, sem1)

    def start(step, buf):
        pltpu.make_async_copy(
            tT_hbm.at[:, pl.ds(step * _TNL, _TNL)], scr.at[buf], sems[buf]
        ).start()

    @pl.when(i == 0)
    def _():
        start(0, 0)
    buf = lax.rem(i, 2)

    def step_body(b):
        @pl.when(i + 1 < pl.num_programs(0))
        def _():
            start(i + 1, 1 - b)

        pltpu.make_async_copy(
            tT_hbm.at[:, pl.ds(i * _TNL, _TNL)], scr.at[b], sems[b]
        ).wait()
        t = scr[b].T
        o_ref[:, pl.ds(0, _EMB)] = t

    @pl.when(buf == 0)
    def _():
        step_body(0)

    @pl.when(buf == 1)
    def _():
        step_body(1)


def _tr_tail_body(tail_ref, big_ref, o_ref):
    del big_ref
    o_ref[:, pl.ds(0, _EMB)] = tail_ref[...].T


def _transpose_table(tableT, tailT):
    main = pl.pallas_call(
        _tr_body,
        grid=(_TSTEPS,),
        in_specs=[pl.BlockSpec(memory_space=pl.ANY)],
        out_specs=pl.BlockSpec((_TNL, 2 * _EMB), lambda i: (i, 0)),
        out_shape=jax.ShapeDtypeStruct((_VOCAB, 2 * _EMB), jnp.float32),
        scratch_shapes=[
            pltpu.VMEM((2, _EMB, _TNL), jnp.float32),
            pltpu.SemaphoreType.DMA,
            pltpu.SemaphoreType.DMA,
        ],
    )(tableT)
    return pl.pallas_call(
        _tr_tail_body,
        grid=(1,),
        in_specs=[
            pl.BlockSpec((_EMB, 64), lambda i: (0, 0)),
            pl.BlockSpec(memory_space=pl.ANY),
        ],
        out_specs=pl.BlockSpec((64, 2 * _EMB), lambda i: (_TMAIN // 64, 0)),
        out_shape=jax.ShapeDtypeStruct((_VOCAB, 2 * _EMB), jnp.float32),
        input_output_aliases={1: 0},
    )(tailT, main)


def _mlp_body(x1_ref, emb_ref, w1_ref, b1_ref, w2_ref, b2_ref, o_ref):
    emb = emb_ref[...] * (1.0 / _L)
    hcat = jnp.concatenate([x1_ref[...], emb], axis=1)
    h = jnp.dot(hcat, w1_ref[...], preferred_element_type=jnp.float32) + b1_ref[...]
    h = jnp.maximum(h, 0.0)
    o = jnp.dot(h, w2_ref[...], preferred_element_type=jnp.float32) + b2_ref[...]
    o_ref[...] = jax.nn.sigmoid(o)


def _mlp(x1, emb_sum, w1, b1, w2, b2):
    blk = 1024
    grid = (_B // blk,)
    return pl.pallas_call(
        _mlp_body,
        grid=grid,
        in_specs=[
            pl.BlockSpec((blk, _DENSE), lambda i: (i, 0)),
            pl.BlockSpec((blk, _EMB), lambda i: (i, 0)),
            pl.BlockSpec((_DENSE + _EMB, _HIDDEN), lambda i: (0, 0)),
            pl.BlockSpec((1, _HIDDEN), lambda i: (0, 0)),
            pl.BlockSpec((_HIDDEN, 1), lambda i: (0, 0)),
            pl.BlockSpec((1, 1), lambda i: (0, 0)),
        ],
        out_specs=pl.BlockSpec((blk, 1), lambda i: (i, 0)),
        out_shape=jax.ShapeDtypeStruct((_B, 1), jnp.float32),
    )(x1, emb_sum, w1, b1, w2, b2)


def kernel(x, table, W1, b1, W2, b2):
    x1 = x[:, :_DENSE]
    idx = x[:, _DENSE:].astype(jnp.int32)
    idx2 = (idx * 2).reshape(_B // _CH, _CH * _L)
    tableT = table.T
    table_wide = _transpose_table(tableT, tableT[:, _TMAIN:])
    emb_sum = _sc_emb(idx2, table_wide.reshape(2 * _VOCAB, _EMB))
    return _mlp(x1, emb_sum, W1, b1.reshape(1, _HIDDEN), W2, b2.reshape(1, 1))


# speedup vs baseline: 4.2390x; 1.0312x over previous
import functools

import jax
import jax.numpy as jnp
from jax import lax
from jax.experimental import pallas as pl
from jax.experimental.pallas import tpu as pltpu
from jax.experimental.pallas import tpu_sc as plsc

_VOCAB = 1000000
_EMB = 64
_HIDDEN = 256
_B = 16384
_L = 50
_DENSE = 64

_NC = 2
_NS = 16
_NW = _NC * _NS
_BPW = _B // _NW
_CH = 2
_NCHUNK = _BPW // _CH
_LANES = 16


_NBUF = 4


def _sc_emb_body(idx_hbm, table_hbm, out_hbm, idx_v, rows_v, out_v,
                 sem0, sem1, sem2, sem3):
    wid = lax.axis_index("s") * _NC + lax.axis_index("c")
    pltpu.sync_copy(idx_hbm.at[pl.ds(wid * _NCHUNK, _NCHUNK)], idx_v)
    sems = (sem0, sem1, sem2, sem3)
    for p in range(_NBUF - 1):
        pltpu.async_copy(table_hbm.at[idx_v.at[p]], rows_v.at[p], sems[p])

    @pl.loop(0, _NCHUNK, step=_NBUF)
    def body(g):
        for b in range(_NBUF):
            i = g + b
            nxt = i + _NBUF - 1
            pb = (b + _NBUF - 1) % _NBUF

            @pl.when(nxt < _NCHUNK)
            def _():
                pltpu.async_copy(
                    table_hbm.at[idx_v.at[nxt]], rows_v.at[pb], sems[pb])

            pltpu.make_async_copy(
                table_hbm.at[idx_v.at[i]], rows_v.at[b], sems[b]).wait()
            for r in range(_CH):
                for jj in range(0, _EMB // _LANES, 2):
                    accs = [rows_v[b, r * _L, pl.ds((jj + j) * _LANES, _LANES)]
                            for j in range(2)]
                    for l in range(1, _L):
                        for j in range(2):
                            accs[j] = accs[j] + rows_v[b, r * _L + l,
                                                       pl.ds((jj + j) * _LANES,
                                                             _LANES)]
                    for j in range(2):
                        out_v[i * _CH + r,
                              pl.ds((jj + j) * _LANES, _LANES)] = accs[j]

    pltpu.sync_copy(out_v, out_hbm.at[pl.ds(wid * _BPW, _BPW)])


@functools.partial(jax.jit, static_argnames=())
def _sc_emb(idx2, table):
    mesh = plsc.VectorSubcoreMesh(core_axis_name="c", subcore_axis_name="s")
    f = functools.partial(
        pl.kernel,
        out_type=jax.ShapeDtypeStruct((_B, _EMB), jnp.float32),
        mesh=mesh,
        scratch_types=[
            pltpu.VMEM((_NCHUNK, _CH * _L), jnp.int32),
            pltpu.VMEM((_NBUF, _CH * _L, _EMB), jnp.float32),
            pltpu.VMEM((_BPW, _EMB), jnp.float32),
            pltpu.SemaphoreType.DMA,
            pltpu.SemaphoreType.DMA,
            pltpu.SemaphoreType.DMA,
            pltpu.SemaphoreType.DMA,
        ],
        compiler_params=pltpu.CompilerParams(use_tc_tiling_on_sc=False),
    )(_sc_emb_body)
    return f(idx2, table)


_TMAIN = 999936
_TNL = 16128
_TSTEPS = _TMAIN // _TNL


def _tr_body(tT_hbm, o_ref, scr, sem0, sem1):
    i = pl.program_id(0)
    sems = (sem0, sem1)

    def start(step, buf):
        pltpu.make_async_copy(
            tT_hbm.at[:, pl.ds(step * _TNL, _TNL)], scr.at[buf], sems[buf]
        ).start()

    @pl.when(i == 0)
    def _():
        start(0, 0)
    buf = lax.rem(i, 2)

    def step_body(b):
        @pl.when(i + 1 < pl.num_programs(0))
        def _():
            start(i + 1, 1 - b)

        pltpu.make_async_copy(
            tT_hbm.at[:, pl.ds(i * _TNL, _TNL)], scr.at[b], sems[b]
        ).wait()
        t = scr[b].T
        o_ref[:, pl.ds(0, _EMB)] = t

    @pl.when(buf == 0)
    def _():
        step_body(0)

    @pl.when(buf == 1)
    def _():
        step_body(1)


def _tr_tail_body(tail_ref, big_ref, o_ref):
    del big_ref
    o_ref[:, pl.ds(0, _EMB)] = tail_ref[...].T


def _transpose_table(tableT, tailT):
    main = pl.pallas_call(
        _tr_body,
        grid=(_TSTEPS,),
        in_specs=[pl.BlockSpec(memory_space=pl.ANY)],
        out_specs=pl.BlockSpec((_TNL, 2 * _EMB), lambda i: (i, 0)),
        out_shape=jax.ShapeDtypeStruct((_VOCAB, 2 * _EMB), jnp.float32),
        scratch_shapes=[
            pltpu.VMEM((2, _EMB, _TNL), jnp.float32),
            pltpu.SemaphoreType.DMA,
            pltpu.SemaphoreType.DMA,
        ],
    )(tableT)
    return pl.pallas_call(
        _tr_tail_body,
        grid=(1,),
        in_specs=[
            pl.BlockSpec((_EMB, 64), lambda i: (0, 0)),
            pl.BlockSpec(memory_space=pl.ANY),
        ],
        out_specs=pl.BlockSpec((64, 2 * _EMB), lambda i: (_TMAIN // 64, 0)),
        out_shape=jax.ShapeDtypeStruct((_VOCAB, 2 * _EMB), jnp.float32),
        input_output_aliases={1: 0},
    )(tailT, main)


def _mlp_body(x1_ref, emb_ref, w1_ref, b1_ref, w2_ref, b2_ref, o_ref):
    emb = emb_ref[...] * (1.0 / _L)
    hcat = jnp.concatenate([x1_ref[...], emb], axis=1)
    h = jnp.dot(hcat, w1_ref[...], preferred_element_type=jnp.float32) + b1_ref[...]
    h = jnp.maximum(h, 0.0)
    o = jnp.dot(h, w2_ref[...], preferred_element_type=jnp.float32) + b2_ref[...]
    o_ref[...] = jax.nn.sigmoid(o)


def _mlp(x1, emb_sum, w1, b1, w2, b2):
    blk = 1024
    grid = (_B // blk,)
    return pl.pallas_call(
        _mlp_body,
        grid=grid,
        in_specs=[
            pl.BlockSpec((blk, _DENSE), lambda i: (i, 0)),
            pl.BlockSpec((blk, _EMB), lambda i: (i, 0)),
            pl.BlockSpec((_DENSE + _EMB, _HIDDEN), lambda i: (0, 0)),
            pl.BlockSpec((1, _HIDDEN), lambda i: (0, 0)),
            pl.BlockSpec((_HIDDEN, 1), lambda i: (0, 0)),
            pl.BlockSpec((1, 1), lambda i: (0, 0)),
        ],
        out_specs=pl.BlockSpec((blk, 1), lambda i: (i, 0)),
        out_shape=jax.ShapeDtypeStruct((_B, 1), jnp.float32),
    )(x1, emb_sum, w1, b1, w2, b2)


def kernel(x, table, W1, b1, W2, b2):
    x1 = x[:, :_DENSE]
    idx = x[:, _DENSE:].astype(jnp.int32)
    idx2 = (idx * 2).reshape(_B // _CH, _CH * _L)
    tableT = table.T
    table_wide = _transpose_table(tableT, tableT[:, _TMAIN:])
    emb_sum = _sc_emb(idx2, table_wide.reshape(2 * _VOCAB, _EMB))
    return _mlp(x1, emb_sum, W1, b1.reshape(1, _HIDDEN), W2, b2.reshape(1, 1))


# speedup vs baseline: 5.0200x; 1.1842x over previous
import functools

import jax
import jax.numpy as jnp
from jax import lax
from jax.experimental import pallas as pl
from jax.experimental.pallas import tpu as pltpu
from jax.experimental.pallas import tpu_sc as plsc

_VOCAB = 1000000
_EMB = 64
_HIDDEN = 256
_B = 16384
_L = 50
_DENSE = 64

_NC = 2
_NS = 16
_NW = _NC * _NS
_BPW = _B // _NW
_CH = 2
_NCHUNK = _BPW // _CH
_LANES = 16


_NBUF = 4


def _sc_emb_body(idx_hbm, table_hbm, out_hbm, idx_v, rows_v, out_v,
                 sem0, sem1, sem2, sem3):
    wid = lax.axis_index("s") * _NC + lax.axis_index("c")
    pltpu.sync_copy(idx_hbm.at[pl.ds(wid * _NCHUNK, _NCHUNK)], idx_v)
    sems = (sem0, sem1, sem2, sem3)
    for p in range(_NBUF - 1):
        pltpu.async_copy(table_hbm.at[idx_v.at[p]], rows_v.at[p], sems[p])

    @pl.loop(0, _NCHUNK, step=_NBUF)
    def body(g):
        for b in range(_NBUF):
            i = g + b
            nxt = i + _NBUF - 1
            pb = (b + _NBUF - 1) % _NBUF

            @pl.when(nxt < _NCHUNK)
            def _():
                pltpu.async_copy(
                    table_hbm.at[idx_v.at[nxt]], rows_v.at[pb], sems[pb])

            pltpu.make_async_copy(
                table_hbm.at[idx_v.at[i]], rows_v.at[b], sems[b]).wait()
            for r in range(_CH):
                for jj in range(0, _EMB // _LANES, 2):
                    accs = [rows_v[b, r * _L, pl.ds((jj + j) * _LANES, _LANES)]
                            for j in range(2)]
                    for l in range(1, _L):
                        for j in range(2):
                            accs[j] = accs[j] + rows_v[b, r * _L + l,
                                                       pl.ds((jj + j) * _LANES,
                                                             _LANES)]
                    for j in range(2):
                        out_v[i * _CH + r,
                              pl.ds((jj + j) * _LANES, _LANES)] = accs[j]

    pltpu.sync_copy(out_v, out_hbm.at[pl.ds(wid * _BPW, _BPW)])


@functools.partial(jax.jit, static_argnames=())
def _sc_emb(idx2, table):
    mesh = plsc.VectorSubcoreMesh(core_axis_name="c", subcore_axis_name="s")
    f = functools.partial(
        pl.kernel,
        out_type=jax.ShapeDtypeStruct((_B, _EMB), jnp.float32),
        mesh=mesh,
        scratch_types=[
            pltpu.VMEM((_NCHUNK, _CH * _L), jnp.int32),
            pltpu.VMEM((_NBUF, _CH * _L, _EMB), jnp.float32),
            pltpu.VMEM((_BPW, _EMB), jnp.float32),
            pltpu.SemaphoreType.DMA,
            pltpu.SemaphoreType.DMA,
            pltpu.SemaphoreType.DMA,
            pltpu.SemaphoreType.DMA,
        ],
        compiler_params=pltpu.CompilerParams(use_tc_tiling_on_sc=False),
    )(_sc_emb_body)
    return f(idx2, table)


_TMAIN = 999936
_TNL = 16128
_TSTEPS = _TMAIN // _TNL


_TH = _TNL // 2


def _tr_body(tT_hbm, o_ref, scr, sem0, sem1):
    i = pl.program_id(0)
    sems = (sem0, sem1)

    def start(step, buf):
        base = step * _TNL
        pltpu.make_async_copy(
            tT_hbm.at[:, pl.ds(base, _TH)],
            scr.at[buf, pl.ds(0, _EMB)], sems[buf]).start()
        pltpu.make_async_copy(
            tT_hbm.at[:, pl.ds(base + _TH, _TH)],
            scr.at[buf, pl.ds(_EMB, _EMB)], sems[buf]).start()

    def wait(step, buf):
        base = step * _TNL
        pltpu.make_async_copy(
            tT_hbm.at[:, pl.ds(base, _TH)],
            scr.at[buf, pl.ds(0, _EMB)], sems[buf]).wait()
        pltpu.make_async_copy(
            tT_hbm.at[:, pl.ds(base + _TH, _TH)],
            scr.at[buf, pl.ds(_EMB, _EMB)], sems[buf]).wait()

    @pl.when(i == 0)
    def _():
        start(0, 0)
    buf = lax.rem(i, 2)

    def step_body(b):
        @pl.when(i + 1 < pl.num_programs(0))
        def _():
            start(i + 1, 1 - b)

        wait(i, b)
        for m in range(_TH // 128):
            t = scr[b, :, pl.ds(m * 128, 128)].T
            o_ref[pl.ds(m * 128, 128), :] = t

    @pl.when(buf == 0)
    def _():
        step_body(0)

    @pl.when(buf == 1)
    def _():
        step_body(1)


def _tr_tail_body(tail_ref, big_ref, o_ref):
    del big_ref
    t = tail_ref[...].T
    o_ref[...] = jnp.concatenate([t[0:32], t[32:64]], axis=1)


def _transpose_table(tableT, tailT):
    main = pl.pallas_call(
        _tr_body,
        grid=(_TSTEPS,),
        in_specs=[pl.BlockSpec(memory_space=pl.ANY)],
        out_specs=pl.BlockSpec((_TH, 2 * _EMB), lambda i: (i, 0)),
        out_shape=jax.ShapeDtypeStruct((_VOCAB // 2, 2 * _EMB), jnp.float32),
        scratch_shapes=[
            pltpu.VMEM((2, 2 * _EMB, _TH), jnp.float32),
            pltpu.SemaphoreType.DMA,
            pltpu.SemaphoreType.DMA,
        ],
    )(tableT)
    return pl.pallas_call(
        _tr_tail_body,
        grid=(1,),
        in_specs=[
            pl.BlockSpec((_EMB, 64), lambda i: (0, 0)),
            pl.BlockSpec(memory_space=pl.ANY),
        ],
        out_specs=pl.BlockSpec((32, 2 * _EMB), lambda i: (_TMAIN // 64, 0)),
        out_shape=jax.ShapeDtypeStruct((_VOCAB // 2, 2 * _EMB), jnp.float32),
        input_output_aliases={1: 0},
    )(tailT, main)


def _mlp_body(x1_ref, emb_ref, w1_ref, b1_ref, w2_ref, b2_ref, o_ref):
    emb = emb_ref[...] * (1.0 / _L)
    hcat = jnp.concatenate([x1_ref[...], emb], axis=1)
    h = jnp.dot(hcat, w1_ref[...], preferred_element_type=jnp.float32) + b1_ref[...]
    h = jnp.maximum(h, 0.0)
    o = jnp.dot(h, w2_ref[...], preferred_element_type=jnp.float32) + b2_ref[...]
    o_ref[...] = jax.nn.sigmoid(o)


def _mlp(x1, emb_sum, w1, b1, w2, b2):
    blk = 1024
    grid = (_B // blk,)
    return pl.pallas_call(
        _mlp_body,
        grid=grid,
        in_specs=[
            pl.BlockSpec((blk, _DENSE), lambda i: (i, 0)),
            pl.BlockSpec((blk, _EMB), lambda i: (i, 0)),
            pl.BlockSpec((_DENSE + _EMB, _HIDDEN), lambda i: (0, 0)),
            pl.BlockSpec((1, _HIDDEN), lambda i: (0, 0)),
            pl.BlockSpec((_HIDDEN, 1), lambda i: (0, 0)),
            pl.BlockSpec((1, 1), lambda i: (0, 0)),
        ],
        out_specs=pl.BlockSpec((blk, 1), lambda i: (i, 0)),
        out_shape=jax.ShapeDtypeStruct((_B, 1), jnp.float32),
    )(x1, emb_sum, w1, b1, w2, b2)


def kernel(x, table, W1, b1, W2, b2):
    x1 = x[:, :_DENSE]
    idx = x[:, _DENSE:].astype(jnp.int32)
    step = idx // _TNL
    loc = idx - step * _TNL
    h = loc // _TH
    m = loc - h * _TH
    g_main = 2 * (step * _TH + m) + h
    rr = idx - _TMAIN
    g_tail = 2 * (_TMAIN // 2 + (rr & 31)) + (rr >> 5)
    g = jnp.where(idx < _TMAIN, g_main, g_tail)
    idx2 = g.reshape(_B // _CH, _CH * _L)
    tableT = table.T
    table_wide = _transpose_table(tableT, tableT[:, _TMAIN:])
    emb_sum = _sc_emb(idx2, table_wide.reshape(_VOCAB, _EMB))
    return _mlp(x1, emb_sum, W1, b1.reshape(1, _HIDDEN), W2, b2.reshape(1, 1))


# speedup vs baseline: 5.1306x; 1.0220x over previous
import functools

import jax
import jax.numpy as jnp
from jax import lax
from jax.experimental import pallas as pl
from jax.experimental.pallas import tpu as pltpu
from jax.experimental.pallas import tpu_sc as plsc

_VOCAB = 1000000
_EMB = 64
_HIDDEN = 256
_B = 16384
_L = 50
_DENSE = 64

_NC = 2
_NS = 16
_NW = _NC * _NS
_BPW = _B // _NW
_CH = 2
_NCHUNK = _BPW // _CH
_LANES = 16


_NBUF = 4


def _sc_emb_body(idx_hbm, table_hbm, out_hbm, idx_v, rows_v, out_v,
                 sem0, sem1, sem2, sem3):
    wid = lax.axis_index("s") * _NC + lax.axis_index("c")
    pltpu.sync_copy(idx_hbm.at[pl.ds(wid * _NCHUNK, _NCHUNK)], idx_v)
    sems = (sem0, sem1, sem2, sem3)
    for p in range(_NBUF - 1):
        pltpu.async_copy(table_hbm.at[idx_v.at[p]], rows_v.at[p], sems[p])

    @pl.loop(0, _NCHUNK, step=_NBUF)
    def body(g):
        for b in range(_NBUF):
            i = g + b
            nxt = i + _NBUF - 1
            pb = (b + _NBUF - 1) % _NBUF

            @pl.when(nxt < _NCHUNK)
            def _():
                pltpu.async_copy(
                    table_hbm.at[idx_v.at[nxt]], rows_v.at[pb], sems[pb])

            pltpu.make_async_copy(
                table_hbm.at[idx_v.at[i]], rows_v.at[b], sems[b]).wait()
            for r in range(_CH):
                for jj in range(0, _EMB // _LANES, 2):
                    accs = [rows_v[b, r * _L, pl.ds((jj + j) * _LANES, _LANES)]
                            for j in range(2)]
                    for l in range(1, _L):
                        for j in range(2):
                            accs[j] = accs[j] + rows_v[b, r * _L + l,
                                                       pl.ds((jj + j) * _LANES,
                                                             _LANES)]
                    for j in range(2):
                        out_v[i * _CH + r,
                              pl.ds((jj + j) * _LANES, _LANES)] = accs[j]

    pltpu.sync_copy(out_v, out_hbm.at[pl.ds(wid * _BPW, _BPW)])


@functools.partial(jax.jit, static_argnames=())
def _sc_emb(idx2, table):
    mesh = plsc.VectorSubcoreMesh(core_axis_name="c", subcore_axis_name="s")
    f = functools.partial(
        pl.kernel,
        out_type=jax.ShapeDtypeStruct((_B, _EMB), jnp.float32),
        mesh=mesh,
        scratch_types=[
            pltpu.VMEM((_NCHUNK, _CH * _L), jnp.int32),
            pltpu.VMEM((_NBUF, _CH * _L, _EMB), jnp.float32),
            pltpu.VMEM((_BPW, _EMB), jnp.float32),
            pltpu.SemaphoreType.DMA,
            pltpu.SemaphoreType.DMA,
            pltpu.SemaphoreType.DMA,
            pltpu.SemaphoreType.DMA,
        ],
        compiler_params=pltpu.CompilerParams(use_tc_tiling_on_sc=False),
    )(_sc_emb_body)
    return f(idx2, table)


_TMAIN = 999936
_HALF = _TMAIN // 2
_TNL = 16128
_TSTEPS = _TMAIN // _TNL


_TH = _TNL // 2


def _tr_body(tT_hbm, o_ref, scr, sem0, sem1):
    i = pl.program_id(0)
    sems = (sem0, sem1)

    def start(step, buf):
        base = step * _TH
        pltpu.make_async_copy(
            tT_hbm.at[:, pl.ds(base, _TH)],
            scr.at[buf, pl.ds(0, _EMB)], sems[buf]).start()
        pltpu.make_async_copy(
            tT_hbm.at[:, pl.ds(_HALF + base, _TH)],
            scr.at[buf, pl.ds(_EMB, _EMB)], sems[buf]).start()

    def wait(step, buf):
        base = step * _TH
        pltpu.make_async_copy(
            tT_hbm.at[:, pl.ds(base, _TH)],
            scr.at[buf, pl.ds(0, _EMB)], sems[buf]).wait()
        pltpu.make_async_copy(
            tT_hbm.at[:, pl.ds(_HALF + base, _TH)],
            scr.at[buf, pl.ds(_EMB, _EMB)], sems[buf]).wait()

    @pl.when(i == 0)
    def _():
        start(0, 0)
    buf = lax.rem(i, 2)

    def step_body(b):
        @pl.when(i + 1 < pl.num_programs(0))
        def _():
            start(i + 1, 1 - b)

        wait(i, b)
        for m in range(_TH // 128):
            t = scr[b, :, pl.ds(m * 128, 128)].T
            o_ref[pl.ds(m * 128, 128), :] = t

    @pl.when(buf == 0)
    def _():
        step_body(0)

    @pl.when(buf == 1)
    def _():
        step_body(1)


def _tr_tail_body(tail_ref, big_ref, o_ref):
    del big_ref
    t = tail_ref[...].T
    o_ref[...] = jnp.concatenate([t[0:32], t[32:64]], axis=1)


def _transpose_table(tableT, tailT):
    main = pl.pallas_call(
        _tr_body,
        grid=(_TSTEPS,),
        in_specs=[pl.BlockSpec(memory_space=pl.ANY)],
        out_specs=pl.BlockSpec((_TH, 2 * _EMB), lambda i: (i, 0)),
        out_shape=jax.ShapeDtypeStruct((_VOCAB // 2, 2 * _EMB), jnp.float32),
        scratch_shapes=[
            pltpu.VMEM((2, 2 * _EMB, _TH), jnp.float32),
            pltpu.SemaphoreType.DMA,
            pltpu.SemaphoreType.DMA,
        ],
    )(tableT)
    return pl.pallas_call(
        _tr_tail_body,
        grid=(1,),
        in_specs=[
            pl.BlockSpec((_EMB, 64), lambda i: (0, 0)),
            pl.BlockSpec(memory_space=pl.ANY),
        ],
        out_specs=pl.BlockSpec((32, 2 * _EMB), lambda i: (_TMAIN // 64, 0)),
        out_shape=jax.ShapeDtypeStruct((_VOCAB // 2, 2 * _EMB), jnp.float32),
        input_output_aliases={1: 0},
    )(tailT, main)


def _mlp_body(x1_ref, emb_ref, w1_ref, b1_ref, w2_ref, b2_ref, o_ref):
    emb = emb_ref[...] * (1.0 / _L)
    hcat = jnp.concatenate([x1_ref[...], emb], axis=1)
    h = jnp.dot(hcat, w1_ref[...], preferred_element_type=jnp.float32) + b1_ref[...]
    h = jnp.maximum(h, 0.0)
    o = jnp.dot(h, w2_ref[...], preferred_element_type=jnp.float32) + b2_ref[...]
    o_ref[...] = jax.nn.sigmoid(o)


def _mlp(x1, emb_sum, w1, b1, w2, b2):
    blk = 2048
    grid = (_B // blk,)
    return pl.pallas_call(
        _mlp_body,
        grid=grid,
        in_specs=[
            pl.BlockSpec((blk, _DENSE), lambda i: (i, 0)),
            pl.BlockSpec((blk, _EMB), lambda i: (i, 0)),
            pl.BlockSpec((_DENSE + _EMB, _HIDDEN), lambda i: (0, 0)),
            pl.BlockSpec((1, _HIDDEN), lambda i: (0, 0)),
            pl.BlockSpec((_HIDDEN, 1), lambda i: (0, 0)),
            pl.BlockSpec((1, 1), lambda i: (0, 0)),
        ],
        out_specs=pl.BlockSpec((blk, 1), lambda i: (i, 0)),
        out_shape=jax.ShapeDtypeStruct((_B, 1), jnp.float32),
    )(x1, emb_sum, w1, b1, w2, b2)


def kernel(x, table, W1, b1, W2, b2):
    x1 = x[:, :_DENSE]
    idx = x[:, _DENSE:].astype(jnp.int32)
    rr = idx - _TMAIN
    g = jnp.where(
        idx < _HALF, 2 * idx,
        jnp.where(idx < _TMAIN, 2 * (idx - _HALF) + 1,
                  2 * (_HALF + (rr & 31)) + (rr >> 5)))
    idx2 = g.reshape(_B // _CH, _CH * _L)
    tableT = table.T
    table_wide = _transpose_table(tableT, tableT[:, _TMAIN:])
    emb_sum = _sc_emb(idx2, table_wide.reshape(_VOCAB, _EMB))
    return _mlp(x1, emb_sum, W1, b1.reshape(1, _HIDDEN), W2, b2.reshape(1, 1))
